# baseline (speedup 1.0000x reference)
import jax
import jax.numpy as jnp
from jax.experimental import pallas as pl
from jax.experimental.pallas import tpu as pltpu
from jax.experimental.pallas import tpu_sc as plsc


_GATHER_WINDOW = 128


def _sc_gather_pairs(table_pairs, idx_phys):
    batch = idx_phys.shape[0]
    width = table_pairs.shape[1]
    idx = idx_phys.reshape(1, batch)
    mesh = plsc.VectorSubcoreMesh(core_axis_name="core",
                                  subcore_axis_name="subcore")

    @pl.kernel(
        out_type=jax.ShapeDtypeStruct((batch, width), table_pairs.dtype),
        mesh=mesh,
    )
    def gather_kernel(table_hbm, idx_hbm, out_hbm):
        def body(idx_vmem, out_vmem):
            pltpu.sync_copy(table_hbm.at[idx_vmem.at[0]], out_vmem)

        pltpu.emit_pipeline(
            body,
            grid=(batch // _GATHER_WINDOW,),
            in_specs=[pl.BlockSpec((1, _GATHER_WINDOW),
                                   index_map=lambda i: (0, i))],
            out_specs=[pl.BlockSpec((_GATHER_WINDOW, width),
                                    index_map=lambda i: (i, 0))],
            core_axis_name=("core", "subcore"),
            dimension_semantics=(pltpu.PARALLEL,),
        )(idx_hbm, out_hbm)

    return gather_kernel(table_pairs, idx)


_VOCAB_TILE = 4096


def _tc_matmul_t(Wt, a):
    embed, vocab = Wt.shape
    batch = a.shape[0]

    def mm_kernel(w_ref, a_ref, o_ref):
        w = w_ref[...].astype(jnp.bfloat16)
        av = a_ref[...].astype(jnp.bfloat16)
        o_ref[...] = jax.lax.dot_general(
            w, av, (((0,), (1,)), ((), ())),
            preferred_element_type=jnp.float32)

    return pl.pallas_call(
        mm_kernel,
        grid=(pl.cdiv(vocab, _VOCAB_TILE),),
        in_specs=[
            pl.BlockSpec((embed, _VOCAB_TILE), lambda i: (0, i)),
            pl.BlockSpec((batch, embed), lambda i: (0, 0)),
        ],
        out_specs=pl.BlockSpec((_VOCAB_TILE, batch), lambda i: (i, 0)),
        out_shape=jax.ShapeDtypeStruct((vocab, batch), jnp.float32),
    )(Wt, a)


def kernel(x, emb_table, W):
    vocab, embed = emb_table.shape
    table_pairs = jnp.concatenate([emb_table[0::2], emb_table[1::2]], axis=1)
    pairs = _sc_gather_pairs(table_pairs, (x >> 1).astype(jnp.int32))
    odd = (x & 1).astype(jnp.bool_).reshape(-1, 1)
    a = jnp.where(odd, pairs[:, embed:], pairs[:, :embed])
    outT = _tc_matmul_t(W.T, a)
    return outT.T


# speedup vs baseline: 1.0048x; 1.0048x over previous
import jax
import jax.numpy as jnp
from jax.experimental import pallas as pl
from jax.experimental.pallas import tpu as pltpu
from jax.experimental.pallas import tpu_sc as plsc


_GATHER_WINDOW = 128


def _sc_gather_pairs(table_pairs, idx_phys):
    batch = idx_phys.shape[0]
    width = table_pairs.shape[1]
    idx = idx_phys.reshape(1, batch)
    mesh = plsc.VectorSubcoreMesh(core_axis_name="core",
                                  subcore_axis_name="subcore")

    @pl.kernel(
        out_type=jax.ShapeDtypeStruct((batch, width), table_pairs.dtype),
        mesh=mesh,
    )
    def gather_kernel(table_hbm, idx_hbm, out_hbm):
        def body(idx_vmem, out_vmem):
            pltpu.sync_copy(table_hbm.at[idx_vmem.at[0]], out_vmem)

        pltpu.emit_pipeline(
            body,
            grid=(batch // _GATHER_WINDOW,),
            in_specs=[pl.BlockSpec((1, _GATHER_WINDOW),
                                   index_map=lambda i: (0, i))],
            out_specs=[pl.BlockSpec((_GATHER_WINDOW, width),
                                    index_map=lambda i: (i, 0))],
            core_axis_name=("core", "subcore"),
            dimension_semantics=(pltpu.PARALLEL,),
        )(idx_hbm, out_hbm)

    return gather_kernel(table_pairs, idx)


_VOCAB_TILE = 2048


def _tc_matmul_t(Wt, a):
    embed, vocab = Wt.shape
    batch = a.shape[0]

    def mm_kernel(w_ref, a_ref, o_ref):
        w = w_ref[...].astype(jnp.bfloat16)
        av = a_ref[...].astype(jnp.bfloat16)
        o_ref[...] = jax.lax.dot_general(
            w, av, (((0,), (1,)), ((), ())),
            preferred_element_type=jnp.float32)

    return pl.pallas_call(
        mm_kernel,
        grid=(pl.cdiv(vocab, _VOCAB_TILE),),
        in_specs=[
            pl.BlockSpec((embed, _VOCAB_TILE), lambda i: (0, i)),
            pl.BlockSpec((batch, embed), lambda i: (0, 0)),
        ],
        out_specs=pl.BlockSpec((_VOCAB_TILE, batch), lambda i: (i, 0)),
        out_shape=jax.ShapeDtypeStruct((vocab, batch), jnp.float32),
    )(Wt, a)


def kernel(x, emb_table, W):
    vocab, embed = emb_table.shape
    table_pairs = jnp.concatenate([emb_table[0::2], emb_table[1::2]], axis=1)
    pairs = _sc_gather_pairs(table_pairs, (x >> 1).astype(jnp.int32))
    odd = (x & 1).astype(jnp.bool_).reshape(-1, 1)
    a = jnp.where(odd, pairs[:, embed:], pairs[:, :embed])
    outT = _tc_matmul_t(W.T, a)
    return outT.T
